# baseline (speedup 1.0000x reference)
import functools

import jax
import jax.numpy as jnp
from jax import lax
from jax.experimental import pallas as pl
from jax.experimental.pallas import tpu as pltpu
from jax.experimental.pallas import tpu_sc as plsc

_NUM_CODES = 1024
_CODE_DIM = 64
_COMMITMENT = 0.25
_TM = 8192


def _dist_argmin_body(flat_ref, emb_ref, idx_ref, acc_ref, *, loss_scale):
    x = flat_ref[...]
    emb = emb_ref[...]
    prod = lax.dot_general(
        x, emb, (((1,), (1,)), ((), ())),
        preferred_element_type=jnp.float32,
        precision=lax.Precision.DEFAULT,
    )
    x2 = jnp.sum(x * x, axis=1, keepdims=True)
    e2 = jnp.sum(emb * emb, axis=1, keepdims=True).T
    dist = x2 - 2.0 * prod + e2
    m = jnp.min(dist, axis=1, keepdims=True)
    ids = lax.broadcasted_iota(jnp.int32, dist.shape, 1)
    idx = jnp.min(jnp.where(dist == m, ids, jnp.int32(2**30)), axis=1)
    idx_ref[...] = idx

    @pl.when(pl.program_id(0) == 0)
    def _():
        acc_ref[...] = jnp.zeros((1, 1), jnp.float32)

    acc_ref[...] += (jnp.sum(m) * loss_scale).reshape(1, 1)


def _dist_argmin(flat, emb, loss_scale, start, n_out):
    grid = n_out // _TM
    step0 = start // _TM
    body = functools.partial(_dist_argmin_body, loss_scale=loss_scale)
    return pl.pallas_call(
        body,
        grid=(grid,),
        in_specs=[
            pl.BlockSpec((_TM, _CODE_DIM), lambda i: (i + step0, 0)),
            pl.BlockSpec((_NUM_CODES, _CODE_DIM), lambda i: (0, 0)),
        ],
        out_specs=[
            pl.BlockSpec((_TM,), lambda i: (i,)),
            pl.BlockSpec((1, 1), lambda i: (0, 0)),
        ],
        out_shape=[
            jax.ShapeDtypeStruct((n_out,), jnp.int32),
            jax.ShapeDtypeStruct((1, 1), jnp.float32),
        ],
    )(flat, emb)


def _make_sc_gather(n_tok, width):
    info = plsc.get_sparse_core_info()
    nc, ns = info.num_cores, info.num_subcores
    nw = nc * ns
    b_per_w = n_tok // nw
    mesh = plsc.VectorSubcoreMesh(core_axis_name="c", subcore_axis_name="s")

    @functools.partial(
        pl.kernel,
        mesh=mesh,
        compiler_params=pltpu.CompilerParams(use_tc_tiling_on_sc=False),
        out_type=jax.ShapeDtypeStruct((n_tok, width), jnp.float32),
        scratch_types=[
            pltpu.VMEM((b_per_w,), jnp.int32),
            pltpu.VMEM((b_per_w, width), jnp.float32),
            pltpu.SemaphoreType.DMA,
        ],
    )
    def gather_k(table_hbm, idx_hbm, out_hbm, idx_v, rows_v, sem):
        wid = lax.axis_index("s") * nc + lax.axis_index("c")
        base = wid * b_per_w
        pltpu.sync_copy(idx_hbm.at[pl.ds(base, b_per_w)], idx_v)
        pltpu.async_copy(table_hbm.at[idx_v], rows_v, sem).wait()
        pltpu.sync_copy(rows_v, out_hbm.at[pl.ds(base, b_per_w)])

    return gather_k


def kernel(z, embedding):
    flat = z.reshape(-1, _CODE_DIM)
    n_tok = flat.shape[0]
    half = n_tok // 2
    scale = _COMMITMENT / (n_tok * _CODE_DIM)
    gather = _make_sc_gather(half, _CODE_DIM)
    idx0, loss0 = _dist_argmin(flat, embedding, scale, 0, half)
    z_q0 = gather(embedding, idx0)
    idx1, loss1 = _dist_argmin(flat, embedding, scale, half, half)
    z_q1 = gather(embedding, idx1)
    z_q = jnp.concatenate([z_q0, z_q1], axis=0)
    indices = jnp.concatenate([idx0, idx1], axis=0)
    loss = (loss0 + loss1).reshape(())
    return z_q.reshape(z.shape), loss, indices


# speedup vs baseline: 1.3620x; 1.3620x over previous
import functools

import jax
import jax.numpy as jnp
from jax import lax
from jax.experimental import pallas as pl
from jax.experimental.pallas import tpu as pltpu
from jax.experimental.pallas import tpu_sc as plsc

_NUM_CODES = 1024
_CODE_DIM = 64
_COMMITMENT = 0.25
_TT = 1024


def _dist_argmin_body(xt_ref, emb_ref, idx_ref, acc_ref, *, loss_scale):
    xt = xt_ref[...]
    emb = emb_ref[...]
    prod = lax.dot_general(
        emb, xt, (((1,), (0,)), ((), ())),
        preferred_element_type=jnp.float32,
        precision=lax.Precision.DEFAULT,
    )
    x2 = jnp.sum(xt * xt, axis=0, keepdims=True)
    e2 = jnp.sum(emb * emb, axis=1, keepdims=True)
    dist = x2 - 2.0 * prod + e2
    m = jnp.min(dist, axis=0, keepdims=True)
    ids = lax.broadcasted_iota(jnp.int32, dist.shape, 0)
    idx = jnp.min(jnp.where(dist == m, ids, jnp.int32(2**30)), axis=0)
    idx_ref[...] = idx.reshape(1, 1, _TT)

    @pl.when(pl.program_id(0) == 0)
    def _():
        acc_ref[...] = jnp.zeros((1, 1), jnp.float32)

    acc_ref[...] += (jnp.sum(m) * loss_scale).reshape(1, 1)


def _dist_argmin(xt2d, emb, loss_scale):
    n_slab = xt2d.shape[0] // _CODE_DIM
    body = functools.partial(_dist_argmin_body, loss_scale=loss_scale)
    return pl.pallas_call(
        body,
        grid=(n_slab,),
        in_specs=[
            pl.BlockSpec((_CODE_DIM, _TT), lambda i: (i, 0)),
            pl.BlockSpec((_NUM_CODES, _CODE_DIM), lambda i: (0, 0)),
        ],
        out_specs=[
            pl.BlockSpec((1, 1, _TT), lambda i: (i, 0, 0)),
            pl.BlockSpec((1, 1), lambda i: (0, 0)),
        ],
        out_shape=[
            jax.ShapeDtypeStruct((n_slab, 1, _TT), jnp.int32),
            jax.ShapeDtypeStruct((1, 1), jnp.float32),
        ],
    )(xt2d, emb)


def _make_sc_gather(n_tok, width):
    info = plsc.get_sparse_core_info()
    nc, ns = info.num_cores, info.num_subcores
    nw = nc * ns
    b_per_w = n_tok // nw
    mesh = plsc.VectorSubcoreMesh(core_axis_name="c", subcore_axis_name="s")

    @functools.partial(
        pl.kernel,
        mesh=mesh,
        compiler_params=pltpu.CompilerParams(use_tc_tiling_on_sc=False),
        out_type=jax.ShapeDtypeStruct((n_tok, width), jnp.float32),
        scratch_types=[
            pltpu.VMEM((b_per_w,), jnp.int32),
            pltpu.VMEM((b_per_w, width), jnp.float32),
            pltpu.SemaphoreType.DMA,
        ],
    )
    def gather_k(table_hbm, idx_hbm, out_hbm, idx_v, rows_v, sem):
        wid = lax.axis_index("s") * nc + lax.axis_index("c")
        base = wid * b_per_w
        pltpu.sync_copy(idx_hbm.at[pl.ds(base, b_per_w)], idx_v)
        pltpu.async_copy(table_hbm.at[idx_v], rows_v, sem).wait()
        pltpu.sync_copy(rows_v, out_hbm.at[pl.ds(base, b_per_w)])

    return gather_k


def kernel(z, embedding):
    n_tok = z.shape[0] * z.shape[1]
    scale = _COMMITMENT / (n_tok * _CODE_DIM)
    xt2d = jnp.transpose(z, (0, 2, 1)).reshape(-1, z.shape[1])
    idx3d, loss2d = _dist_argmin(xt2d, embedding, scale)
    indices = idx3d.reshape(n_tok)
    z_q = _make_sc_gather(n_tok, _CODE_DIM)(embedding, indices)
    return z_q.reshape(z.shape), loss2d.reshape(()), indices


# speedup vs baseline: 1.4035x; 1.0305x over previous
import functools

import jax
import jax.numpy as jnp
from jax import lax
from jax.experimental import pallas as pl
from jax.experimental.pallas import tpu as pltpu
from jax.experimental.pallas import tpu_sc as plsc

_NUM_CODES = 1024
_CODE_DIM = 64
_COMMITMENT = 0.25
_TT = 1024


def _dist_argmin_body(xt_ref, emb_ref, idx_ref, acc_ref, *, loss_scale):
    xt = xt_ref[...]
    emb = emb_ref[...]
    prod = lax.dot_general(
        emb, xt, (((1,), (0,)), ((), ())),
        preferred_element_type=jnp.float32,
        precision=lax.Precision.DEFAULT,
    )
    x2 = jnp.sum(xt * xt, axis=0, keepdims=True)
    e2 = jnp.sum(emb * emb, axis=1, keepdims=True)
    dist = x2 - 2.0 * prod + e2
    m = jnp.min(dist, axis=0, keepdims=True)
    ids = lax.broadcasted_iota(jnp.int32, dist.shape, 0)
    idx = jnp.min(jnp.where(dist == m, ids, jnp.int32(2**30)), axis=0)
    idx_ref[...] = idx.reshape(1, 1, _TT)

    @pl.when(pl.program_id(0) == 0)
    def _():
        acc_ref[...] = jnp.zeros((1, 1), jnp.float32)

    acc_ref[...] += (jnp.sum(m) * loss_scale).reshape(1, 1)


def _dist_argmin(xt2d, emb, loss_scale):
    n_slab = xt2d.shape[0] // _CODE_DIM
    body = functools.partial(_dist_argmin_body, loss_scale=loss_scale)
    return pl.pallas_call(
        body,
        grid=(n_slab,),
        in_specs=[
            pl.BlockSpec((_CODE_DIM, _TT), lambda i: (i, 0)),
            pl.BlockSpec((_NUM_CODES, _CODE_DIM), lambda i: (0, 0)),
        ],
        out_specs=[
            pl.BlockSpec((1, 1, _TT), lambda i: (i, 0, 0)),
            pl.BlockSpec((1, 1), lambda i: (0, 0)),
        ],
        out_shape=[
            jax.ShapeDtypeStruct((n_slab, 1, _TT), jnp.int32),
            jax.ShapeDtypeStruct((1, 1), jnp.float32),
        ],
    )(xt2d, emb)


def _make_sc_gather_t(n_tok, width):
    info = plsc.get_sparse_core_info()
    nc, ns, nl = info.num_cores, info.num_subcores, info.num_lanes
    nw = nc * ns
    b_per_w = n_tok // nw
    d_half = width // 2
    mesh = plsc.VectorSubcoreMesh(core_axis_name="c", subcore_axis_name="s")

    @functools.partial(
        pl.kernel,
        mesh=mesh,
        compiler_params=pltpu.CompilerParams(
            use_tc_tiling_on_sc=False, needs_layout_passes=False),
        out_type=jax.ShapeDtypeStruct((nw * width, b_per_w), jnp.float32),
        scratch_types=[
            pltpu.VMEM((width, _NUM_CODES), jnp.float32),
            pltpu.VMEM((b_per_w,), jnp.int32),
            pltpu.VMEM((d_half, b_per_w), jnp.float32),
        ],
    )
    def gather_k(tab_hbm, idx_hbm, out_hbm, tab_v, idx_v, out_v):
        wid = lax.axis_index("s") * nc + lax.axis_index("c")
        pltpu.sync_copy(tab_hbm, tab_v)
        pltpu.sync_copy(idx_hbm.at[pl.ds(wid * b_per_w, b_per_w)], idx_v)
        for p in range(2):

            def tok_body(t, _):
                iv = idx_v[pl.ds(t * nl, nl)]
                for d in range(d_half):
                    row = jnp.full((nl,), p * d_half + d, jnp.int32)
                    out_v[d, pl.ds(t * nl, nl)] = plsc.load_gather(
                        tab_v, [row, iv])
                return 0

            lax.fori_loop(0, b_per_w // nl, tok_body, 0, unroll=False)
            pltpu.sync_copy(
                out_v, out_hbm.at[pl.ds(wid * width + p * d_half, d_half)])

    return gather_k


def kernel(z, embedding):
    n_tok = z.shape[0] * z.shape[1]
    scale = _COMMITMENT / (n_tok * _CODE_DIM)
    xt2d = jnp.transpose(z, (0, 2, 1)).reshape(-1, z.shape[1])
    idx3d, loss2d = _dist_argmin(xt2d, embedding, scale)
    indices = idx3d.reshape(n_tok)
    embt = jnp.transpose(embedding)
    zq_t = _make_sc_gather_t(n_tok, _CODE_DIM)(embt, indices)
    z_q = zq_t.reshape(z.shape[0], _CODE_DIM, z.shape[1]).transpose(0, 2, 1)
    return z_q, loss2d.reshape(()), indices


# speedup vs baseline: 1.4321x; 1.0203x over previous
import functools

import jax
import jax.numpy as jnp
from jax import lax
from jax.experimental import pallas as pl
from jax.experimental.pallas import tpu as pltpu
from jax.experimental.pallas import tpu_sc as plsc

_NUM_CODES = 1024
_CODE_DIM = 64
_COMMITMENT = 0.25
_TT = 1024


def _dist_argmin_body(xt_ref, emb_ref, idx_ref, acc_ref, *, loss_scale):
    xt = xt_ref[...]
    emb = emb_ref[...]
    prod = lax.dot_general(
        emb, xt, (((1,), (0,)), ((), ())),
        preferred_element_type=jnp.float32,
        precision=lax.Precision.DEFAULT,
    )
    x2 = jnp.sum(xt * xt, axis=0, keepdims=True)
    e2 = jnp.sum(emb * emb, axis=1, keepdims=True)
    dist = x2 - 2.0 * prod + e2
    m = jnp.min(dist, axis=0, keepdims=True)
    ids = lax.broadcasted_iota(jnp.int32, dist.shape, 0)
    idx = jnp.min(jnp.where(dist == m, ids, jnp.int32(2**30)), axis=0)
    idx_ref[...] = idx.reshape(1, 1, _TT)

    @pl.when(pl.program_id(0) == 0)
    def _():
        acc_ref[...] = jnp.zeros((1, 1), jnp.float32)

    acc_ref[...] += (jnp.sum(m) * loss_scale).reshape(1, 1)


def _dist_argmin(xt2d, emb, loss_scale):
    n_slab = xt2d.shape[0] // _CODE_DIM
    body = functools.partial(_dist_argmin_body, loss_scale=loss_scale)
    return pl.pallas_call(
        body,
        grid=(n_slab,),
        in_specs=[
            pl.BlockSpec((_CODE_DIM, _TT), lambda i: (i, 0)),
            pl.BlockSpec((_NUM_CODES, _CODE_DIM), lambda i: (0, 0)),
        ],
        out_specs=[
            pl.BlockSpec((1, 1, _TT), lambda i: (i, 0, 0)),
            pl.BlockSpec((1, 1), lambda i: (0, 0)),
        ],
        out_shape=[
            jax.ShapeDtypeStruct((n_slab, 1, _TT), jnp.int32),
            jax.ShapeDtypeStruct((1, 1), jnp.float32),
        ],
    )(xt2d, emb)


def _make_sc_gather_t(n_tok, width):
    info = plsc.get_sparse_core_info()
    nc, ns, nl = info.num_cores, info.num_subcores, info.num_lanes
    nw = nc * ns
    b_per_w = n_tok // nw
    d_half = width // 2
    mesh = plsc.VectorSubcoreMesh(core_axis_name="c", subcore_axis_name="s")

    n_pass = 4
    d_pp = width // n_pass

    @functools.partial(
        pl.kernel,
        mesh=mesh,
        compiler_params=pltpu.CompilerParams(
            use_tc_tiling_on_sc=False, needs_layout_passes=False),
        out_type=jax.ShapeDtypeStruct((nw * width, b_per_w), jnp.float32),
        scratch_types=[
            pltpu.VMEM((width, _NUM_CODES), jnp.float32),
            pltpu.VMEM((b_per_w,), jnp.int32),
            pltpu.VMEM((d_pp, b_per_w), jnp.float32),
            pltpu.VMEM((d_pp, b_per_w), jnp.float32),
            pltpu.SemaphoreType.DMA,
            pltpu.SemaphoreType.DMA,
            pltpu.SemaphoreType.DMA,
        ],
    )
    def gather_k(tab_hbm, idx_hbm, out_hbm, tab_v, idx_v, o0, o1, si, s0, s1):
        wid = lax.axis_index("s") * nc + lax.axis_index("c")
        ht = pltpu.async_copy(tab_hbm, tab_v, si)
        hi = pltpu.async_copy(
            idx_hbm.at[pl.ds(wid * b_per_w, b_per_w)], idx_v, si)
        ht.wait()
        hi.wait()
        outs, sems = (o0, o1), (s0, s1)
        waits = [None] * n_pass
        for p in range(n_pass):
            if p >= 2:
                waits[p - 2].wait()
            out_v = outs[p % 2]

            def tok_body(t, _, p=p, out_v=out_v):
                iv = idx_v[pl.ds(t * nl, nl)]
                for d in range(d_pp):
                    row = jnp.full((nl,), p * d_pp + d, jnp.int32)
                    out_v[d, pl.ds(t * nl, nl)] = plsc.load_gather(
                        tab_v, [row, iv])
                return 0

            lax.fori_loop(0, b_per_w // nl, tok_body, 0, unroll=False)
            waits[p] = pltpu.async_copy(
                out_v, out_hbm.at[pl.ds(wid * width + p * d_pp, d_pp)],
                sems[p % 2])
        waits[n_pass - 2].wait()
        waits[n_pass - 1].wait()

    return gather_k


def kernel(z, embedding):
    n_tok = z.shape[0] * z.shape[1]
    scale = _COMMITMENT / (n_tok * _CODE_DIM)
    xt2d = jnp.transpose(z, (0, 2, 1)).reshape(-1, z.shape[1])
    idx3d, loss2d = _dist_argmin(xt2d, embedding, scale)
    indices = idx3d.reshape(n_tok)
    embt = jnp.transpose(embedding)
    zq_t = _make_sc_gather_t(n_tok, _CODE_DIM)(embt, indices)
    z_q = zq_t.reshape(z.shape[0], _CODE_DIM, z.shape[1]).transpose(0, 2, 1)
    return z_q, loss2d.reshape(()), indices
